# baseline (speedup 1.0000x reference)
import functools

import jax
import jax.numpy as jnp
from jax import lax
from jax.experimental import pallas as pl
from jax.experimental.pallas import tpu as pltpu
from jax.experimental.pallas import tpu_sc as plsc

NC, NS, NL = 2, 16, 16
NW = NC * NS

M = 384


def _router_kernel(x_ref, wr_ref, iw_ref, ww_ref):
    E = 8
    xb = x_ref[...]
    logits = jnp.dot(xb, wr_ref[...], preferred_element_type=jnp.float32)
    bt = logits.shape[0]
    lane = lax.broadcasted_iota(jnp.int32, (bt, 128), 1)
    neg = jnp.float32(-1e30)
    logits = jnp.where(lane < E, logits, neg)
    m1 = jnp.max(logits, axis=1, keepdims=True)
    i1 = jnp.min(jnp.where(logits == m1, lane, 2**30), axis=1, keepdims=True)
    l2 = jnp.where(lane == i1, neg, logits)
    m2 = jnp.max(l2, axis=1, keepdims=True)
    i2 = jnp.min(jnp.where(l2 == m2, lane, 2**30), axis=1, keepdims=True)
    w1 = jax.nn.sigmoid(m1 - m2)
    w2 = 1.0 - w1
    iw_ref[...] = jnp.where(lane == 0, i1, jnp.where(lane == 1, i2, 0))
    ww_ref[...] = jnp.where(lane < 16, w1, jnp.where(lane < 32, w2, 0.0))


def _router(x, Wr):
    T, D = x.shape
    BT = 512
    wr_pad = jnp.zeros((D, 128), jnp.float32).at[:, : Wr.shape[1]].set(Wr)
    iw, ww = pl.pallas_call(
        _router_kernel,
        grid=(T // BT,),
        in_specs=[
            pl.BlockSpec((BT, D), lambda t: (t, 0)),
            pl.BlockSpec((D, 128), lambda t: (0, 0)),
        ],
        out_specs=[
            pl.BlockSpec((BT, 128), lambda t: (t, 0)),
            pl.BlockSpec((BT, 128), lambda t: (t, 0)),
        ],
        out_shape=[
            jax.ShapeDtypeStruct((T, 128), jnp.int32),
            jax.ShapeDtypeStruct((T, 128), jnp.float32),
        ],
    )(x, wr_pad)
    return iw[:, :2], ww


def _grouping(idx, E, NT):
    i_flat = idx.reshape(-1)
    oh = (i_flat[:, None] == jnp.arange(E, dtype=jnp.int32)[None, :]).astype(jnp.int32)
    cum = jnp.cumsum(oh, axis=0)
    rank = jnp.sum(oh * cum, axis=1) - 1
    counts = cum[-1]
    tiles_pe = (counts + M - 1) // M
    tile_start = jnp.concatenate(
        [jnp.zeros((1,), jnp.int32), jnp.cumsum(tiles_pe)[:-1].astype(jnp.int32)]
    )
    pos = jnp.sum(oh * (tile_start * M)[None, :], axis=1) + rank
    tile_ids = jnp.arange(NT, dtype=jnp.int32)
    te = jnp.sum((tile_ids[:, None] >= tile_start[None, :]).astype(jnp.int32), axis=1) - 1
    valid = (tile_ids < jnp.sum(tiles_pe)).astype(jnp.int32)
    return te, valid, pos


def _make_dispatch(T, P, D):
    TPW = T // NW
    mesh = plsc.VectorSubcoreMesh(core_axis_name="c", subcore_axis_name="s")

    @functools.partial(
        pl.kernel,
        mesh=mesh,
        out_type=jax.ShapeDtypeStruct((P, D), jnp.float32),
        scratch_types=[
            pltpu.VMEM((TPW,), jnp.int32),
            pltpu.VMEM((TPW,), jnp.int32),
            pltpu.VMEM((TPW, D), jnp.float32),
            pltpu.SemaphoreType.DMA,
        ],
    )
    def dispatch_k(x_hbm, pe_hbm, po_hbm, out_hbm, pe_v, po_v, rows_v, sem):
        wid = lax.axis_index("s") * NC + lax.axis_index("c")
        base = wid * TPW
        pltpu.sync_copy(pe_hbm.at[pl.ds(base, TPW)], pe_v)
        pltpu.sync_copy(po_hbm.at[pl.ds(base, TPW)], po_v)
        pltpu.sync_copy(x_hbm.at[pl.ds(base, TPW)], rows_v)
        ca = pltpu.async_copy(rows_v, out_hbm.at[pe_v], sem)
        cb = pltpu.async_copy(rows_v, out_hbm.at[po_v], sem)
        ca.wait()
        cb.wait()

    return dispatch_k


def _mlp_kernel(te_ref, valid_ref, x_ref, wg_ref, wu_ref, wd_ref, o_ref):
    t = pl.program_id(0)

    @pl.when(valid_ref[t] != 0)
    def _():
        xb = x_ref[...]
        g = jnp.dot(xb, wg_ref[0], preferred_element_type=jnp.float32)
        u = jnp.dot(xb, wu_ref[0], preferred_element_type=jnp.float32)
        h = g * jax.nn.sigmoid(g) * u
        o_ref[...] = jnp.dot(h, wd_ref[0], preferred_element_type=jnp.float32)


def _grouped_mlp(te, valid, xg, Wg, Wu, Wd, NT):
    P, D = xg.shape
    F = Wg.shape[2]
    grid_spec = pltpu.PrefetchScalarGridSpec(
        num_scalar_prefetch=2,
        grid=(NT,),
        in_specs=[
            pl.BlockSpec((M, D), lambda t, te_r, v_r: (t, 0)),
            pl.BlockSpec((1, D, F), lambda t, te_r, v_r: (te_r[t], 0, 0)),
            pl.BlockSpec((1, D, F), lambda t, te_r, v_r: (te_r[t], 0, 0)),
            pl.BlockSpec((1, F, D), lambda t, te_r, v_r: (te_r[t], 0, 0)),
        ],
        out_specs=pl.BlockSpec((M, D), lambda t, te_r, v_r: (t, 0)),
    )
    return pl.pallas_call(
        _mlp_kernel,
        grid_spec=grid_spec,
        out_shape=jax.ShapeDtypeStruct((P, D), jnp.float32),
    )(te, valid, xg, Wg, Wu, Wd)


def _make_combine(T, P, D):
    TPW = T // NW
    CH = 32
    NCH = TPW // CH
    mesh = plsc.VectorSubcoreMesh(core_axis_name="c", subcore_axis_name="s")

    @functools.partial(
        pl.kernel,
        mesh=mesh,
        out_type=jax.ShapeDtypeStruct((T, D), jnp.float32),
        scratch_types=[
            pltpu.VMEM((TPW,), jnp.int32),
            pltpu.VMEM((TPW,), jnp.int32),
            pltpu.VMEM((TPW, 128), jnp.float32),
            pltpu.VMEM((CH, D), jnp.float32),
            pltpu.VMEM((CH, D), jnp.float32),
            pltpu.VMEM((CH, D), jnp.float32),
            pltpu.VMEM((CH, D), jnp.float32),
            pltpu.SemaphoreType.DMA,
        ],
    )
    def combine_k(y_hbm, p0_hbm, p1_hbm, ww_hbm, out_hbm,
                  p0_v, p1_v, w_v, a0_v, a1_v, b0_v, b1_v, sem):
        a_bufs = [a0_v, a1_v]
        b_bufs = [b0_v, b1_v]
        wid = lax.axis_index("s") * NC + lax.axis_index("c")
        pltpu.sync_copy(p0_hbm.at[pl.ds(wid * TPW, TPW)], p0_v)
        pltpu.sync_copy(p1_hbm.at[pl.ds(wid * TPW, TPW)], p1_v)
        pltpu.sync_copy(ww_hbm.at[pl.ds(wid * TPW, TPW)], w_v)

        def gstart(c):
            sl = pl.ds(c * CH, CH)
            return (
                pltpu.async_copy(y_hbm.at[p0_v.at[sl]], a_bufs[c % 2], sem),
                pltpu.async_copy(y_hbm.at[p1_v.at[sl]], b_bufs[c % 2], sem),
            )

        cps = {0: gstart(0)}
        for c in range(NCH):
            if c + 1 < NCH:
                cps[(c + 1) % 2] = gstart(c + 1)
            ca, cb = cps[c % 2]
            ca.wait()
            cb.wait()
            a_v = a_bufs[c % 2]
            b_v = b_bufs[c % 2]

            def row(r, _):
                wa = w_v[c * CH + r, pl.ds(0, NL)]
                wb = w_v[c * CH + r, pl.ds(NL, NL)]
                for cc in range(D // NL):
                    a_v[r, pl.ds(cc * NL, NL)] = (
                        a_v[r, pl.ds(cc * NL, NL)] * wa
                        + b_v[r, pl.ds(cc * NL, NL)] * wb
                    )
                return 0

            lax.fori_loop(0, CH, row, 0)
            pltpu.sync_copy(a_v, out_hbm.at[pl.ds(wid * TPW + c * CH, CH)])

    return combine_k


def kernel(x, Wr, Wg, Wu, Wd):
    T, D = x.shape
    E = Wr.shape[1]
    NT = (2 * T) // M + E
    P = NT * M

    iw, ww = _router(x, Wr)
    te, valid, pos = _grouping(iw, E, NT)

    pos2 = pos.reshape(T, 2)
    dispatch_k = _make_dispatch(T, P, D)
    xg = dispatch_k(x, pos2[:, 0], pos2[:, 1])

    y = _grouped_mlp(te, valid, xg, Wg, Wu, Wd, NT)

    combine_k = _make_combine(T, P, D)
    return combine_k(y, pos2[:, 0], pos2[:, 1], ww)


# speedup vs baseline: 1.2063x; 1.2063x over previous
import functools

import jax
import jax.numpy as jnp
from jax import lax
from jax.experimental import pallas as pl
from jax.experimental.pallas import tpu as pltpu
from jax.experimental.pallas import tpu_sc as plsc

NC, NS, NL = 2, 16, 16
NW = NC * NS

M = 640


def _router_kernel(x_ref, wr_ref, iw_ref, ww_ref):
    E = 8
    xb = x_ref[...]
    logits = jnp.dot(xb, wr_ref[...], preferred_element_type=jnp.float32)
    bt = logits.shape[0]
    lane = lax.broadcasted_iota(jnp.int32, (bt, 128), 1)
    neg = jnp.float32(-1e30)
    logits = jnp.where(lane < E, logits, neg)
    m1 = jnp.max(logits, axis=1, keepdims=True)
    i1 = jnp.min(jnp.where(logits == m1, lane, 2**30), axis=1, keepdims=True)
    l2 = jnp.where(lane == i1, neg, logits)
    m2 = jnp.max(l2, axis=1, keepdims=True)
    i2 = jnp.min(jnp.where(l2 == m2, lane, 2**30), axis=1, keepdims=True)
    w1 = jax.nn.sigmoid(m1 - m2)
    w2 = 1.0 - w1
    iw_ref[...] = jnp.where(lane == 0, i1, jnp.where(lane == 1, i2, 0))
    ww_ref[...] = jnp.where(lane < 16, w1, jnp.where(lane < 32, w2, 0.0))


def _router(x, Wr):
    T, D = x.shape
    BT = 512
    wr_pad = jnp.zeros((D, 128), jnp.float32).at[:, : Wr.shape[1]].set(Wr)
    iw, ww = pl.pallas_call(
        _router_kernel,
        grid=(T // BT,),
        in_specs=[
            pl.BlockSpec((BT, D), lambda t: (t, 0)),
            pl.BlockSpec((D, 128), lambda t: (0, 0)),
        ],
        out_specs=[
            pl.BlockSpec((BT, 128), lambda t: (t, 0)),
            pl.BlockSpec((BT, 128), lambda t: (t, 0)),
        ],
        out_shape=[
            jax.ShapeDtypeStruct((T, 128), jnp.int32),
            jax.ShapeDtypeStruct((T, 128), jnp.float32),
        ],
    )(x, wr_pad)
    return iw[:, :2], ww


def _grouping(idx, E, NT):
    i_flat = idx.reshape(-1)
    oh = (i_flat[:, None] == jnp.arange(E, dtype=jnp.int32)[None, :]).astype(jnp.int32)
    cum = jnp.cumsum(oh, axis=0)
    rank = jnp.sum(oh * cum, axis=1) - 1
    counts = cum[-1]
    tiles_pe = (counts + M - 1) // M
    tile_start = jnp.concatenate(
        [jnp.zeros((1,), jnp.int32), jnp.cumsum(tiles_pe)[:-1].astype(jnp.int32)]
    )
    pos = jnp.sum(oh * (tile_start * M)[None, :], axis=1) + rank
    tile_ids = jnp.arange(NT, dtype=jnp.int32)
    te = jnp.sum((tile_ids[:, None] >= tile_start[None, :]).astype(jnp.int32), axis=1) - 1
    valid = (tile_ids < jnp.sum(tiles_pe)).astype(jnp.int32)
    return te, valid, pos


def _make_dispatch(T, P, D):
    TPW = T // NW
    mesh = plsc.VectorSubcoreMesh(core_axis_name="c", subcore_axis_name="s")

    @functools.partial(
        pl.kernel,
        mesh=mesh,
        out_type=jax.ShapeDtypeStruct((P, D), jnp.float32),
        scratch_types=[
            pltpu.VMEM((TPW,), jnp.int32),
            pltpu.VMEM((TPW,), jnp.int32),
            pltpu.VMEM((TPW, D), jnp.float32),
            pltpu.SemaphoreType.DMA,
        ],
    )
    def dispatch_k(x_hbm, pe_hbm, po_hbm, out_hbm, pe_v, po_v, rows_v, sem):
        wid = lax.axis_index("s") * NC + lax.axis_index("c")
        base = wid * TPW
        pltpu.sync_copy(pe_hbm.at[pl.ds(base, TPW)], pe_v)
        pltpu.sync_copy(po_hbm.at[pl.ds(base, TPW)], po_v)
        pltpu.sync_copy(x_hbm.at[pl.ds(base, TPW)], rows_v)
        ca = pltpu.async_copy(rows_v, out_hbm.at[pe_v], sem)
        cb = pltpu.async_copy(rows_v, out_hbm.at[po_v], sem)
        ca.wait()
        cb.wait()

    return dispatch_k


def _mlp_kernel(te_ref, valid_ref, x_ref, wg_ref, wu_ref, wd_ref, o_ref):
    t = pl.program_id(0)

    @pl.when(valid_ref[t] != 0)
    def _():
        xb = x_ref[...]
        g = jnp.dot(xb, wg_ref[0], preferred_element_type=jnp.float32)
        u = jnp.dot(xb, wu_ref[0], preferred_element_type=jnp.float32)
        h = g * jax.nn.sigmoid(g) * u
        o_ref[...] = jnp.dot(h, wd_ref[0], preferred_element_type=jnp.float32)


def _grouped_mlp(te, valid, xg, Wg, Wu, Wd, NT):
    P, D = xg.shape
    F = Wg.shape[2]
    grid_spec = pltpu.PrefetchScalarGridSpec(
        num_scalar_prefetch=2,
        grid=(NT,),
        in_specs=[
            pl.BlockSpec((M, D), lambda t, te_r, v_r: (t, 0)),
            pl.BlockSpec((1, D, F), lambda t, te_r, v_r: (te_r[t], 0, 0)),
            pl.BlockSpec((1, D, F), lambda t, te_r, v_r: (te_r[t], 0, 0)),
            pl.BlockSpec((1, F, D), lambda t, te_r, v_r: (te_r[t], 0, 0)),
        ],
        out_specs=pl.BlockSpec((M, D), lambda t, te_r, v_r: (t, 0)),
    )
    return pl.pallas_call(
        _mlp_kernel,
        grid_spec=grid_spec,
        out_shape=jax.ShapeDtypeStruct((P, D), jnp.float32),
    )(te, valid, xg, Wg, Wu, Wd)


def _make_combine(T, P, D):
    TPW = T // NW
    CH = 32
    NCH = TPW // CH
    mesh = plsc.VectorSubcoreMesh(core_axis_name="c", subcore_axis_name="s")

    @functools.partial(
        pl.kernel,
        mesh=mesh,
        out_type=jax.ShapeDtypeStruct((T, D), jnp.float32),
        scratch_types=[
            pltpu.VMEM((TPW,), jnp.int32),
            pltpu.VMEM((TPW,), jnp.int32),
            pltpu.VMEM((TPW, 128), jnp.float32),
            pltpu.VMEM((CH, D), jnp.float32),
            pltpu.VMEM((CH, D), jnp.float32),
            pltpu.VMEM((CH, D), jnp.float32),
            pltpu.VMEM((CH, D), jnp.float32),
            pltpu.SemaphoreType.DMA,
        ],
    )
    def combine_k(y_hbm, p0_hbm, p1_hbm, ww_hbm, out_hbm,
                  p0_v, p1_v, w_v, a0_v, a1_v, b0_v, b1_v, sem):
        a_bufs = [a0_v, a1_v]
        b_bufs = [b0_v, b1_v]
        wid = lax.axis_index("s") * NC + lax.axis_index("c")
        pltpu.sync_copy(p0_hbm.at[pl.ds(wid * TPW, TPW)], p0_v)
        pltpu.sync_copy(p1_hbm.at[pl.ds(wid * TPW, TPW)], p1_v)
        pltpu.sync_copy(ww_hbm.at[pl.ds(wid * TPW, TPW)], w_v)

        def gstart(c):
            sl = pl.ds(c * CH, CH)
            return (
                pltpu.async_copy(y_hbm.at[p0_v.at[sl]], a_bufs[c % 2], sem),
                pltpu.async_copy(y_hbm.at[p1_v.at[sl]], b_bufs[c % 2], sem),
            )

        cps = {0: gstart(0)}
        for c in range(NCH):
            if c + 1 < NCH:
                cps[(c + 1) % 2] = gstart(c + 1)
            ca, cb = cps[c % 2]
            ca.wait()
            cb.wait()
            a_v = a_bufs[c % 2]
            b_v = b_bufs[c % 2]

            def row(r, _):
                wa = w_v[c * CH + r, pl.ds(0, NL)]
                wb = w_v[c * CH + r, pl.ds(NL, NL)]
                for cc in range(D // NL):
                    a_v[r, pl.ds(cc * NL, NL)] = (
                        a_v[r, pl.ds(cc * NL, NL)] * wa
                        + b_v[r, pl.ds(cc * NL, NL)] * wb
                    )
                return 0

            lax.fori_loop(0, CH, row, 0)
            pltpu.sync_copy(a_v, out_hbm.at[pl.ds(wid * TPW + c * CH, CH)])

    return combine_k


def kernel(x, Wr, Wg, Wu, Wd):
    T, D = x.shape
    E = Wr.shape[1]
    NT = (2 * T) // M + E
    P = NT * M

    iw, ww = _router(x, Wr)
    te, valid, pos = _grouping(iw, E, NT)

    pos2 = pos.reshape(T, 2)
    dispatch_k = _make_dispatch(T, P, D)
    xg = dispatch_k(x, pos2[:, 0], pos2[:, 1])

    y = _grouped_mlp(te, valid, xg, Wg, Wu, Wd, NT)

    combine_k = _make_combine(T, P, D)
    return combine_k(y, pos2[:, 0], pos2[:, 1], ww)
